# baseline (speedup 1.0000x reference)
import functools

import jax
import jax.numpy as jnp
from jax import lax
from jax.experimental import pallas as pl
from jax.experimental.pallas import tpu as pltpu
from jax.experimental.pallas import tpu_sc as plsc

NUM_CORES = 2
NUM_SUBCORES = 16
CHUNK = 128
ROWS_PER_BLOCK = 8
DST_BITS = 15
DMASK = (1 << DST_BITS) - 1
DUMMY_SPREAD = 64


def _round_up(v, m):
    return ((v + m - 1) // m) * m


@functools.lru_cache(maxsize=None)
def _make_sc_segment_sum(half_rows, d, k2):
    acc_rows = half_rows + CHUNK
    rpt = acc_rows // NUM_SUBCORES
    tail = half_rows - (NUM_SUBCORES - 1) * rpt
    assert half_rows % CHUNK == 0 and rpt % 8 == 0 and 0 < tail <= rpt
    assert k2 % ROWS_PER_BLOCK == 0
    mesh = plsc.VectorSubcoreMesh(core_axis_name="c", subcore_axis_name="s")

    @functools.partial(
        pl.kernel,
        mesh=mesh,
        out_type=jax.ShapeDtypeStruct((NUM_CORES * half_rows, d), jnp.float32),
        scratch_types=[
            pltpu.VMEM((2 * ROWS_PER_BLOCK, CHUNK), jnp.int32),
            pltpu.VMEM((1, CHUNK), jnp.int32),
            pltpu.VMEM((2, CHUNK), jnp.int32),
            pltpu.VMEM((2, CHUNK), jnp.int32),
            pltpu.VMEM((CHUNK, d), jnp.float32),
            pltpu.VMEM((CHUNK, d), jnp.float32),
            pltpu.VMEM((rpt, d), jnp.float32),
            pltpu.VMEM_SHARED((acc_rows, d), jnp.float32),
            pltpu.SemaphoreType.DMA,
            pltpu.SemaphoreType.DMA,
            pltpu.SemaphoreType.DMA,
            pltpu.SemaphoreType.DMA,
            pltpu.SemaphoreType.DMA,
        ],
    )
    def ksum(segs_h, cnts_h, table_h, out_h,
             stg, cbuf, sidx, didx, bufa, bufb, zbuf, acc,
             gsa, gsb, ssa, ssb, lsem):
        c = lax.axis_index("c")
        s = lax.axis_index("s")
        lo = c * half_rows
        base = s * rpt
        region = (c * NUM_SUBCORES + s) * k2

        pltpu.sync_copy(cnts_h.at[pl.ds(8 * c, 1)], cbuf)
        nch_core = cbuf[0, pl.ds(0, 16)][0]
        nloc = jnp.clip((nch_core - s + NUM_SUBCORES - 1) // NUM_SUBCORES,
                        0, k2)
        nblk = (nloc + ROWS_PER_BLOCK - 1) // ROWS_PER_BLOCK

        def stg_load(b, half):
            pltpu.async_copy(
                segs_h.at[pl.ds(region + b * ROWS_PER_BLOCK,
                                ROWS_PER_BLOCK)],
                stg.at[pl.ds(half * ROWS_PER_BLOCK, ROWS_PER_BLOCK)], lsem)

        def stg_wait(half):
            pltpu.make_async_copy(
                segs_h.at[pl.ds(0, ROWS_PER_BLOCK)],
                stg.at[pl.ds(half * ROWS_PER_BLOCK, ROWS_PER_BLOCK)],
                lsem).wait()

        @pl.when(nblk > 0)
        def _():
            stg_load(0, 0)

        def zero_row(i, carry):
            for kk in range(d // 16):
                zbuf[i, pl.ds(kk * 16, 16)] = jnp.zeros((16,), jnp.float32)
            return carry

        lax.fori_loop(0, rpt, zero_row, 0)
        pltpu.sync_copy(zbuf, acc.at[pl.ds(base, rpt)])
        plsc.subcore_barrier()

        def unpack(hrow, row):
            for kk in range(CHUNK // 16):
                v = stg[hrow, pl.ds(kk * 16, 16)]
                dglob = v & DMASK
                dl = dglob - lo
                own = (dl >= 0) & (dl < half_rows)
                dummy = half_rows + (dglob & (DUMMY_SPREAD - 1))
                didx[row, pl.ds(kk * 16, 16)] = jnp.where(own, dl, dummy)
                sidx[row, pl.ds(kk * 16, 16)] = v >> DST_BITS

        def step(wait_cond, hrow, buf, gsem, ssem, row):
            if wait_cond is None:
                pltpu.make_async_copy(buf, acc.at[didx.at[row]], ssem).wait()
            else:
                @pl.when(wait_cond)
                def _():
                    pltpu.make_async_copy(
                        buf, acc.at[didx.at[row]], ssem).wait()

            unpack(hrow, row)
            pltpu.async_copy(table_h.at[sidx.at[row]], buf, gsem)

        def block(b, carry):
            stg_wait(b % 2)

            @pl.when(b + 1 < nblk)
            def _():
                stg_load(b + 1, (b + 1) % 2)

            hoff = (b % 2) * ROWS_PER_BLOCK
            for p in range(ROWS_PER_BLOCK // 2):
                wait0 = (b > 0) if p == 0 else None
                step(wait0, hoff + 2 * p, bufa, gsa, ssa, 0)
                step(wait0, hoff + 2 * p + 1, bufb, gsb, ssb, 1)
                pltpu.make_async_copy(table_h.at[sidx.at[0]], bufa, gsa).wait()
                pltpu.async_copy(bufa, acc.at[didx.at[0]], ssa, add=True)
                pltpu.make_async_copy(table_h.at[sidx.at[1]], bufb, gsb).wait()
                pltpu.async_copy(bufb, acc.at[didx.at[1]], ssb, add=True)
            return carry

        lax.fori_loop(0, nblk, block, 0)

        @pl.when(nblk > 0)
        def _():
            pltpu.make_async_copy(bufa, acc.at[didx.at[0]], ssa).wait()
            pltpu.make_async_copy(bufb, acc.at[didx.at[1]], ssb).wait()

        plsc.subcore_barrier()

        gbase = c * half_rows + base

        @pl.when(s < NUM_SUBCORES - 1)
        def _():
            pltpu.sync_copy(acc.at[pl.ds(base, rpt)], zbuf)
            pltpu.sync_copy(zbuf, out_h.at[pl.ds(gbase, rpt)])

        @pl.when(s == NUM_SUBCORES - 1)
        def _():
            pltpu.sync_copy(acc.at[pl.ds(base, tail)], zbuf.at[pl.ds(0, tail)])
            pltpu.sync_copy(zbuf.at[pl.ds(0, tail)],
                            out_h.at[pl.ds(gbase, tail)])

    return ksum


def _stage_a(x, w1cat, br):
    n, nf = x.shape
    hc = w1cat.shape[1] // 2
    d1 = _round_up(hc + 1, 128)

    def body(x_ref, w_ref, t1_ref, xr_ref):
        o = jnp.dot(x_ref[...], w_ref[...], preferred_element_type=jnp.float32)
        xw = o[:, :hc]
        pad = jnp.concatenate(
            [jnp.ones((o.shape[0], 1), jnp.float32),
             jnp.zeros((o.shape[0], d1 - hc - 1), jnp.float32)], axis=1)
        t1_ref[...] = jnp.concatenate([xw, pad], axis=1)
        xr_ref[...] = o[:, hc:]

    return pl.pallas_call(
        body,
        grid=(n // br,),
        in_specs=[
            pl.BlockSpec((br, nf), lambda i: (i, 0)),
            pl.BlockSpec((nf, 2 * hc), lambda i: (0, 0)),
        ],
        out_specs=[
            pl.BlockSpec((br, d1), lambda i: (i, 0)),
            pl.BlockSpec((br, hc), lambda i: (i, 0)),
        ],
        out_shape=[
            jax.ShapeDtypeStruct((n, d1), jnp.float32),
            jax.ShapeDtypeStruct((n, hc), jnp.float32),
        ],
    )(x, w1cat)


def _stage_c(p1, xr1, b1, w2cat, br):
    n, hc = xr1.shape
    d1 = p1.shape[1]
    nc = w2cat.shape[1] // 2

    def body(p_ref, xr_ref, b1_ref, w_ref, emb_ref, inv_ref, t2_ref, hr_ref):
        ssum = p_ref[...]
        cnt = ssum[:, hc:hc + 1]
        inv = 1.0 / jnp.maximum(cnt, 1.0)
        emb = ssum[:, :hc] * inv + xr_ref[...] + b1_ref[...][None, :]
        h = jnp.maximum(emb, 0.0)
        o = jnp.dot(h, w_ref[...], preferred_element_type=jnp.float32)
        emb_ref[...] = emb
        inv_ref[...] = inv
        t2_ref[...] = o[:, :nc]
        hr_ref[...] = o[:, nc:]

    return pl.pallas_call(
        body,
        grid=(n // br,),
        in_specs=[
            pl.BlockSpec((br, d1), lambda i: (i, 0)),
            pl.BlockSpec((br, hc), lambda i: (i, 0)),
            pl.BlockSpec((hc,), lambda i: (0,)),
            pl.BlockSpec((hc, 2 * nc), lambda i: (0, 0)),
        ],
        out_specs=[
            pl.BlockSpec((br, hc), lambda i: (i, 0)),
            pl.BlockSpec((br, 1), lambda i: (i, 0)),
            pl.BlockSpec((br, nc), lambda i: (i, 0)),
            pl.BlockSpec((br, nc), lambda i: (i, 0)),
        ],
        out_shape=[
            jax.ShapeDtypeStruct((n, hc), jnp.float32),
            jax.ShapeDtypeStruct((n, 1), jnp.float32),
            jax.ShapeDtypeStruct((n, nc), jnp.float32),
            jax.ShapeDtypeStruct((n, nc), jnp.float32),
        ],
    )(p1, xr1, b1, w2cat)


def _stage_e(p2, hr2, inv, b2, br):
    n, nc = hr2.shape

    def body(p_ref, hr_ref, inv_ref, b2_ref, out_ref):
        out_ref[...] = (p_ref[...] * inv_ref[...] + hr_ref[...]
                        + b2_ref[...][None, :])

    return pl.pallas_call(
        body,
        grid=(n // br,),
        in_specs=[
            pl.BlockSpec((br, nc), lambda i: (i, 0)),
            pl.BlockSpec((br, nc), lambda i: (i, 0)),
            pl.BlockSpec((br, 1), lambda i: (i, 0)),
            pl.BlockSpec((nc,), lambda i: (0,)),
        ],
        out_specs=pl.BlockSpec((br, nc), lambda i: (i, 0)),
        out_shape=jax.ShapeDtypeStruct((n, nc), jnp.float32),
    )(p2, hr2, inv, b2)


def kernel(x, edge_index, w1, root1, b1, w2, root2, b2):
    n, nf = x.shape
    e = edge_index.shape[1]
    hc = w1.shape[2]
    nc = w2.shape[2]
    br = 1000

    half_rows = _round_up(-(-n // 2), CHUNK)
    blk_edges = NUM_SUBCORES * ROWS_PER_BLOCK * CHUNK
    cap_seg = _round_up(e, blk_edges)
    k2 = cap_seg // (NUM_SUBCORES * CHUNK)

    src = edge_index[0]
    dst = edge_index[1]
    packed = src * (1 << DST_BITS) + dst
    own0 = dst < half_rows
    pos0 = jnp.cumsum(own0.astype(jnp.int32)) - 1
    pos1 = jnp.cumsum(1 - own0.astype(jnp.int32)) - 1
    idx = jnp.where(own0, pos0, cap_seg + pos1)
    segs_flat = jnp.full((2 * cap_seg,), DMASK, jnp.int32)
    segs_flat = segs_flat.at[idx].set(packed, unique_indices=True)
    segs = (segs_flat.reshape(2, k2, NUM_SUBCORES, CHUNK)
            .transpose(0, 2, 1, 3).reshape(-1, CHUNK))
    n0 = jnp.sum(own0.astype(jnp.int32))
    nch0 = -(-n0 // CHUNK)
    nch1 = -(-(e - n0) // CHUNK)
    cnts = jnp.concatenate(
        [jnp.full((8, CHUNK), nch0, jnp.int32),
         jnp.full((8, CHUNK), nch1, jnp.int32)])

    w1cat = jnp.concatenate([w1[0], root1], axis=1)
    w2cat = jnp.concatenate([w2[0], root2], axis=1)

    t1, xr1 = _stage_a(x, w1cat, br)
    d1 = t1.shape[1]
    seg_fn = _make_sc_segment_sum(half_rows, d1, k2)

    def body(carry):
        i, table, emb, inv, hr2, logits = carry
        p = seg_fn(segs, cnts, table)

        def f0(_):
            emb2, inv2, t2, hr22 = _stage_c(p[:n], xr1, b1, w2cat, br)
            return (t2, emb2, inv2, hr22, logits)

        def f1(_):
            lg = _stage_e(p[:n], hr2, inv, b2, br)
            return (table, emb, inv, hr2, lg)

        table, emb, inv, hr2, logits = lax.cond(i == 0, f0, f1, None)
        return (i + 1, table, emb, inv, hr2, logits)

    init = (jnp.int32(0), t1,
            jnp.zeros((n, hc), jnp.float32),
            jnp.zeros((n, 1), jnp.float32),
            jnp.zeros((n, nc), jnp.float32),
            jnp.zeros((n, nc), jnp.float32))
    _, _, emb, _, _, logits = lax.while_loop(lambda c: c[0] < 2, body, init)
    return (logits, emb)


# speedup vs baseline: 2.0817x; 2.0817x over previous
import functools

import jax
import jax.numpy as jnp
from jax import lax
from jax.experimental import pallas as pl
from jax.experimental.pallas import tpu as pltpu
from jax.experimental.pallas import tpu_sc as plsc

NUM_CORES = 2
NUM_SUBCORES = 16
CHUNK = 128
ROWS_PER_BLOCK = 8
DST_BITS = 15
DMASK = (1 << DST_BITS) - 1
DUMMY_SPREAD = 64


def _round_up(v, m):
    return ((v + m - 1) // m) * m


@functools.lru_cache(maxsize=None)
def _make_sc_segment_sum(half_rows, d, k2):
    acc_rows = half_rows + CHUNK
    rpt = acc_rows // NUM_SUBCORES
    tail = half_rows - (NUM_SUBCORES - 1) * rpt
    assert half_rows % CHUNK == 0 and rpt % 8 == 0 and 0 < tail <= rpt
    assert k2 % ROWS_PER_BLOCK == 0
    mesh = plsc.VectorSubcoreMesh(core_axis_name="c", subcore_axis_name="s")

    @functools.partial(
        pl.kernel,
        mesh=mesh,
        out_type=jax.ShapeDtypeStruct((NUM_CORES * half_rows, d), jnp.float32),
        scratch_types=[
            pltpu.VMEM((2 * ROWS_PER_BLOCK, CHUNK), jnp.int32),
            pltpu.VMEM((1, CHUNK), jnp.int32),
            pltpu.VMEM((2, CHUNK), jnp.int32),
            pltpu.VMEM((2, CHUNK), jnp.int32),
            pltpu.VMEM((CHUNK, d), jnp.float32),
            pltpu.VMEM((CHUNK, d), jnp.float32),
            pltpu.VMEM((rpt, d), jnp.float32),
            pltpu.VMEM_SHARED((acc_rows, d), jnp.float32),
            pltpu.SemaphoreType.DMA,
            pltpu.SemaphoreType.DMA,
            pltpu.SemaphoreType.DMA,
            pltpu.SemaphoreType.DMA,
            pltpu.SemaphoreType.DMA,
        ],
    )
    def ksum(segs_h, cnts_h, table_h, out_h,
             stg, cbuf, sidx, didx, bufa, bufb, zbuf, acc,
             gsa, gsb, ssa, ssb, lsem):
        c = lax.axis_index("c")
        s = lax.axis_index("s")
        lo = c * half_rows
        base = s * rpt
        region = (c * NUM_SUBCORES + s) * k2

        pltpu.sync_copy(cnts_h.at[pl.ds(8 * c, 1)], cbuf)
        nch_core = cbuf[0, pl.ds(0, 16)][0]
        nloc = jnp.clip((nch_core - s + NUM_SUBCORES - 1) // NUM_SUBCORES,
                        0, k2)
        nblk = (nloc + ROWS_PER_BLOCK - 1) // ROWS_PER_BLOCK

        def stg_load(b, half):
            pltpu.async_copy(
                segs_h.at[pl.ds(region + b * ROWS_PER_BLOCK,
                                ROWS_PER_BLOCK)],
                stg.at[pl.ds(half * ROWS_PER_BLOCK, ROWS_PER_BLOCK)], lsem)

        def stg_wait(half):
            pltpu.make_async_copy(
                segs_h.at[pl.ds(0, ROWS_PER_BLOCK)],
                stg.at[pl.ds(half * ROWS_PER_BLOCK, ROWS_PER_BLOCK)],
                lsem).wait()

        @pl.when(nblk > 0)
        def _():
            stg_load(0, 0)

        def zero_row(i, carry):
            for kk in range(d // 16):
                zbuf[i, pl.ds(kk * 16, 16)] = jnp.zeros((16,), jnp.float32)
            return carry

        lax.fori_loop(0, rpt, zero_row, 0)
        pltpu.sync_copy(zbuf, acc.at[pl.ds(base, rpt)])
        plsc.subcore_barrier()

        def unpack(hrow, row):
            for kk in range(CHUNK // 16):
                v = stg[hrow, pl.ds(kk * 16, 16)]
                dglob = v & DMASK
                dl = dglob - lo
                own = (dl >= 0) & (dl < half_rows)
                dummy = half_rows + (dglob & (DUMMY_SPREAD - 1))
                didx[row, pl.ds(kk * 16, 16)] = jnp.where(own, dl, dummy)
                sidx[row, pl.ds(kk * 16, 16)] = v >> DST_BITS

        def step(wait_cond, hrow, buf, gsem, ssem, row):
            if wait_cond is None:
                pltpu.make_async_copy(buf, acc.at[didx.at[row]], ssem).wait()
            else:
                @pl.when(wait_cond)
                def _():
                    pltpu.make_async_copy(
                        buf, acc.at[didx.at[row]], ssem).wait()

            unpack(hrow, row)
            pltpu.async_copy(table_h.at[sidx.at[row]], buf, gsem)

        def block(b, carry):
            stg_wait(b % 2)

            @pl.when(b + 1 < nblk)
            def _():
                stg_load(b + 1, (b + 1) % 2)

            hoff = (b % 2) * ROWS_PER_BLOCK
            for p in range(ROWS_PER_BLOCK // 2):
                wait0 = (b > 0) if p == 0 else None
                step(wait0, hoff + 2 * p, bufa, gsa, ssa, 0)
                step(wait0, hoff + 2 * p + 1, bufb, gsb, ssb, 1)
                pltpu.make_async_copy(table_h.at[sidx.at[0]], bufa, gsa).wait()
                pltpu.async_copy(bufa, acc.at[didx.at[0]], ssa, add=True)
                pltpu.make_async_copy(table_h.at[sidx.at[1]], bufb, gsb).wait()
                pltpu.async_copy(bufb, acc.at[didx.at[1]], ssb, add=True)
            return carry

        lax.fori_loop(0, nblk, block, 0)

        @pl.when(nblk > 0)
        def _():
            pltpu.make_async_copy(bufa, acc.at[didx.at[0]], ssa).wait()
            pltpu.make_async_copy(bufb, acc.at[didx.at[1]], ssb).wait()

        plsc.subcore_barrier()

        gbase = c * half_rows + base

        @pl.when(s < NUM_SUBCORES - 1)
        def _():
            pltpu.sync_copy(acc.at[pl.ds(base, rpt)], zbuf)
            pltpu.sync_copy(zbuf, out_h.at[pl.ds(gbase, rpt)])

        @pl.when(s == NUM_SUBCORES - 1)
        def _():
            pltpu.sync_copy(acc.at[pl.ds(base, tail)], zbuf.at[pl.ds(0, tail)])
            pltpu.sync_copy(zbuf.at[pl.ds(0, tail)],
                            out_h.at[pl.ds(gbase, tail)])

    return ksum


def _stage_a(x, w1cat, br):
    n, nf = x.shape
    hc = w1cat.shape[1] // 2
    d1 = _round_up(hc + 1, 128)

    def body(x_ref, w_ref, t1_ref, xr_ref):
        o = jnp.dot(x_ref[...], w_ref[...], preferred_element_type=jnp.float32)
        xw = o[:, :hc]
        pad = jnp.concatenate(
            [jnp.ones((o.shape[0], 1), jnp.float32),
             jnp.zeros((o.shape[0], d1 - hc - 1), jnp.float32)], axis=1)
        t1_ref[...] = jnp.concatenate([xw, pad], axis=1)
        xr_ref[...] = o[:, hc:]

    return pl.pallas_call(
        body,
        grid=(n // br,),
        in_specs=[
            pl.BlockSpec((br, nf), lambda i: (i, 0)),
            pl.BlockSpec((nf, 2 * hc), lambda i: (0, 0)),
        ],
        out_specs=[
            pl.BlockSpec((br, d1), lambda i: (i, 0)),
            pl.BlockSpec((br, hc), lambda i: (i, 0)),
        ],
        out_shape=[
            jax.ShapeDtypeStruct((n, d1), jnp.float32),
            jax.ShapeDtypeStruct((n, hc), jnp.float32),
        ],
    )(x, w1cat)


def _stage_c(p1, xr1, b1, w2cat, br):
    n, hc = xr1.shape
    d1 = p1.shape[1]
    nc = w2cat.shape[1] // 2

    def body(p_ref, xr_ref, b1_ref, w_ref, emb_ref, inv_ref, t2_ref, hr_ref):
        ssum = p_ref[...]
        cnt = ssum[:, hc:hc + 1]
        inv = 1.0 / jnp.maximum(cnt, 1.0)
        emb = ssum[:, :hc] * inv + xr_ref[...] + b1_ref[...][None, :]
        h = jnp.maximum(emb, 0.0)
        o = jnp.dot(h, w_ref[...], preferred_element_type=jnp.float32)
        emb_ref[...] = emb
        inv_ref[...] = inv
        t2_ref[...] = o[:, :nc]
        hr_ref[...] = o[:, nc:]

    return pl.pallas_call(
        body,
        grid=(n // br,),
        in_specs=[
            pl.BlockSpec((br, d1), lambda i: (i, 0)),
            pl.BlockSpec((br, hc), lambda i: (i, 0)),
            pl.BlockSpec((hc,), lambda i: (0,)),
            pl.BlockSpec((hc, 2 * nc), lambda i: (0, 0)),
        ],
        out_specs=[
            pl.BlockSpec((br, hc), lambda i: (i, 0)),
            pl.BlockSpec((br, 1), lambda i: (i, 0)),
            pl.BlockSpec((br, nc), lambda i: (i, 0)),
            pl.BlockSpec((br, nc), lambda i: (i, 0)),
        ],
        out_shape=[
            jax.ShapeDtypeStruct((n, hc), jnp.float32),
            jax.ShapeDtypeStruct((n, 1), jnp.float32),
            jax.ShapeDtypeStruct((n, nc), jnp.float32),
            jax.ShapeDtypeStruct((n, nc), jnp.float32),
        ],
    )(p1, xr1, b1, w2cat)


def _stage_e(p2, hr2, inv, b2, br):
    n, nc = hr2.shape

    def body(p_ref, hr_ref, inv_ref, b2_ref, out_ref):
        out_ref[...] = (p_ref[...] * inv_ref[...] + hr_ref[...]
                        + b2_ref[...][None, :])

    return pl.pallas_call(
        body,
        grid=(n // br,),
        in_specs=[
            pl.BlockSpec((br, nc), lambda i: (i, 0)),
            pl.BlockSpec((br, nc), lambda i: (i, 0)),
            pl.BlockSpec((br, 1), lambda i: (i, 0)),
            pl.BlockSpec((nc,), lambda i: (0,)),
        ],
        out_specs=pl.BlockSpec((br, nc), lambda i: (i, 0)),
        out_shape=jax.ShapeDtypeStruct((n, nc), jnp.float32),
    )(p2, hr2, inv, b2)


def kernel(x, edge_index, w1, root1, b1, w2, root2, b2):
    n, nf = x.shape
    e = edge_index.shape[1]
    hc = w1.shape[2]
    nc = w2.shape[2]
    br = 1000

    half_rows = _round_up(-(-n // 2), CHUNK)
    blk_edges = NUM_SUBCORES * ROWS_PER_BLOCK * CHUNK
    cap_seg = _round_up(e, blk_edges)
    k2 = cap_seg // (NUM_SUBCORES * CHUNK)

    src = edge_index[0]
    dst = edge_index[1]
    packed = src * (1 << DST_BITS) + dst
    own0 = dst < half_rows
    pos0 = jnp.cumsum(own0.astype(jnp.int32)) - 1
    pos1 = jnp.arange(e, dtype=jnp.int32) - pos0 - 1
    idx = jnp.where(own0, pos0, cap_seg + pos1)
    segs_flat = jnp.zeros((2 * cap_seg,), jnp.int32)
    segs_flat = segs_flat.at[idx].add(packed + 1, unique_indices=True)
    segs_flat = jnp.where(segs_flat == 0, DMASK, segs_flat - 1)
    segs = (segs_flat.reshape(2, k2, NUM_SUBCORES, CHUNK)
            .transpose(0, 2, 1, 3).reshape(-1, CHUNK))
    n0 = jnp.sum(own0.astype(jnp.int32))
    nch0 = -(-n0 // CHUNK)
    nch1 = -(-(e - n0) // CHUNK)
    cnts = jnp.concatenate(
        [jnp.full((8, CHUNK), nch0, jnp.int32),
         jnp.full((8, CHUNK), nch1, jnp.int32)])

    w1cat = jnp.concatenate([w1[0], root1], axis=1)
    w2cat = jnp.concatenate([w2[0], root2], axis=1)

    t1, xr1 = _stage_a(x, w1cat, br)
    d1 = t1.shape[1]
    seg_fn = _make_sc_segment_sum(half_rows, d1, k2)

    def body(carry):
        i, table, emb, inv, hr2, logits = carry
        p = seg_fn(segs, cnts, table)

        def f0(_):
            emb2, inv2, t2, hr22 = _stage_c(p[:n], xr1, b1, w2cat, br)
            return (t2, emb2, inv2, hr22, logits)

        def f1(_):
            lg = _stage_e(p[:n], hr2, inv, b2, br)
            return (table, emb, inv, hr2, lg)

        table, emb, inv, hr2, logits = lax.cond(i == 0, f0, f1, None)
        return (i + 1, table, emb, inv, hr2, logits)

    init = (jnp.int32(0), t1,
            jnp.zeros((n, hc), jnp.float32),
            jnp.zeros((n, 1), jnp.float32),
            jnp.zeros((n, nc), jnp.float32),
            jnp.zeros((n, nc), jnp.float32))
    _, _, emb, _, _, logits = lax.while_loop(lambda c: c[0] < 2, body, init)
    return (logits, emb)
